# Initial kernel scaffold; baseline (speedup 1.0000x reference)
import jax
import jax.numpy as jnp
from jax.experimental import pallas as pl


def kernel(feat, edge_index, W1, b1, W2, b2):
    raise NotImplementedError("write your pallas kernel here")


# speedup vs baseline: 5.1408x; 5.1408x over previous
import functools

import jax
import jax.numpy as jnp
from jax import lax
from jax.experimental import pallas as pl
from jax.experimental.pallas import tpu as pltpu
from jax.experimental.pallas import tpu_sc as plsc

N = 10000
D = 128
E = 320000

NC = 2
NS = 16
NW = NC * NS

NPAD = 10240
PADROW = NPAD - 1
EPT = E // NW
CHUNK = 128
NCH = -(-EPT // CHUNK)
EPT_PAD = NCH * CHUNK
RPT = NPAD // NS
DEGW = 16

BLK = 1024
GRID = NPAD // BLK


def _mesh():
    return plsc.VectorSubcoreMesh(core_axis_name="c", subcore_axis_name="s")


def _deg_call(src, dst):
    @functools.partial(
        pl.kernel,
        mesh=_mesh(),
        out_type=jax.ShapeDtypeStruct((NC, 2, NPAD), jnp.float32),
        scratch_types=[
            pltpu.VMEM_SHARED((NPAD,), jnp.float32),
            pltpu.VMEM_SHARED((NPAD,), jnp.float32),
            pltpu.VMEM((NCH, CHUNK), jnp.int32),
            pltpu.VMEM((NCH, CHUNK), jnp.int32),
            pltpu.VMEM((CHUNK,), jnp.float32),
            pltpu.VMEM((RPT,), jnp.float32),
        ],
    )
    def k(src_hbm, dst_hbm, out_hbm, acc_s, acc_d, isrc, idst, ones, zbuf):
        c = lax.axis_index("c")
        s = lax.axis_index("s")
        wid = s * NC + c
        off = s * RPT

        def fill_ones(r, _):
            ones[pl.ds(r * 16, 16)] = jnp.ones((16,), jnp.float32)
            return 0

        lax.fori_loop(0, CHUNK // 16, fill_ones, 0)

        def fill_zero(r, _):
            zbuf[pl.ds(r * 16, 16)] = jnp.zeros((16,), jnp.float32)
            return 0

        lax.fori_loop(0, RPT // 16, fill_zero, 0)

        pltpu.sync_copy(zbuf, acc_s.at[pl.ds(off, RPT)])
        pltpu.sync_copy(zbuf, acc_d.at[pl.ds(off, RPT)])
        pltpu.sync_copy(src_hbm.at[wid], isrc)
        pltpu.sync_copy(dst_hbm.at[wid], idst)
        plsc.subcore_barrier()

        def body(j, _):
            pltpu.sync_copy(ones, acc_s.at[isrc.at[j]], add=True)
            pltpu.sync_copy(ones, acc_d.at[idst.at[j]], add=True)
            return 0

        lax.fori_loop(0, NCH, body, 0)
        plsc.subcore_barrier()
        pltpu.sync_copy(acc_s.at[pl.ds(off, RPT)], zbuf)
        pltpu.sync_copy(zbuf, out_hbm.at[c, 0, pl.ds(off, RPT)])
        pltpu.sync_copy(acc_d.at[pl.ds(off, RPT)], zbuf)
        pltpu.sync_copy(zbuf, out_hbm.at[c, 1, pl.ds(off, RPT)])

    return k(src, dst)


def _agg_call(h, src, dst):
    @functools.partial(
        pl.kernel,
        mesh=_mesh(),
        out_type=jax.ShapeDtypeStruct((NC, NPAD, D), jnp.float32),
        scratch_types=[
            pltpu.VMEM_SHARED((NPAD, D), jnp.float32),
            pltpu.VMEM((NCH, CHUNK), jnp.int32),
            pltpu.VMEM((NCH, CHUNK), jnp.int32),
            pltpu.VMEM((CHUNK, D), jnp.float32),
        ],
    )
    def k(h_hbm, src_hbm, dst_hbm, out_hbm, acc, isrc, idst, rbuf):
        c = lax.axis_index("c")
        s = lax.axis_index("s")
        wid = s * NC + c
        off = s * RPT

        def fill_zero(i, _):
            r = i // (D // 16)
            col = (i % (D // 16)) * 16
            rbuf[r, pl.ds(col, 16)] = jnp.zeros((16,), jnp.float32)
            return 0

        lax.fori_loop(0, CHUNK * (D // 16), fill_zero, 0)
        for t in range(RPT // CHUNK):
            pltpu.sync_copy(rbuf, acc.at[pl.ds(off + t * CHUNK, CHUNK)])
        pltpu.sync_copy(src_hbm.at[wid], isrc)
        pltpu.sync_copy(dst_hbm.at[wid], idst)
        plsc.subcore_barrier()

        def body(j, _):
            pltpu.sync_copy(h_hbm.at[isrc.at[j]], rbuf)
            pltpu.sync_copy(rbuf, acc.at[idst.at[j]], add=True)
            return 0

        lax.fori_loop(0, NCH, body, 0)
        plsc.subcore_barrier()

        def copy_out(t, _):
            pltpu.sync_copy(acc.at[pl.ds(off + t * CHUNK, CHUNK)], rbuf)
            pltpu.sync_copy(rbuf, out_hbm.at[c, pl.ds(off + t * CHUNK, CHUNK)])
            return 0

        lax.fori_loop(0, RPT // CHUNK, copy_out, 0)

    return k(h, src, dst)


def _norms_matmul_call(degp, feat, W1):
    def body(degp_ref, feat_ref, w_ref, h_ref, ns_ref, nd_ref):
        x = degp_ref[...]
        d_o = x[0, 0, :, 0:1] + x[1, 0, :, 0:1]
        d_i = x[0, 1, :, 0:1] + x[1, 1, :, 0:1]
        ns = lax.rsqrt(jnp.where(d_o > 0, d_o, 1.0))
        nd = lax.rsqrt(jnp.where(d_i > 0, d_i, 1.0))
        ns_ref[...] = ns
        nd_ref[...] = nd
        h_ref[...] = jnp.dot(feat_ref[...] * ns, w_ref[...],
                             preferred_element_type=jnp.float32)

    return pl.pallas_call(
        body,
        grid=(GRID,),
        in_specs=[
            pl.BlockSpec((NC, 2, BLK, 1), lambda i: (0, 0, i, 0)),
            pl.BlockSpec((BLK, D), lambda i: (i, 0)),
            pl.BlockSpec((D, D), lambda i: (0, 0)),
        ],
        out_specs=[
            pl.BlockSpec((BLK, D), lambda i: (i, 0)),
            pl.BlockSpec((BLK, 1), lambda i: (i, 0)),
            pl.BlockSpec((BLK, 1), lambda i: (i, 0)),
        ],
        out_shape=[
            jax.ShapeDtypeStruct((NPAD, D), jnp.float32),
            jax.ShapeDtypeStruct((NPAD, 1), jnp.float32),
            jax.ShapeDtypeStruct((NPAD, 1), jnp.float32),
        ],
    )(degp, feat, W1)


def _mid_call(p, nd, ns, b1, W2):
    def body(p_ref, nd_ref, ns_ref, b_ref, w_ref, h_ref):
        agg = p_ref[0] + p_ref[1]
        o = jnp.maximum(agg * nd_ref[...] + b_ref[...], 0.0)
        h_ref[...] = jnp.dot(o * ns_ref[...], w_ref[...],
                             preferred_element_type=jnp.float32)

    return pl.pallas_call(
        body,
        grid=(GRID,),
        in_specs=[
            pl.BlockSpec((NC, BLK, D), lambda i: (0, i, 0)),
            pl.BlockSpec((BLK, 1), lambda i: (i, 0)),
            pl.BlockSpec((BLK, 1), lambda i: (i, 0)),
            pl.BlockSpec((1, D), lambda i: (0, 0)),
            pl.BlockSpec((D, D), lambda i: (0, 0)),
        ],
        out_specs=pl.BlockSpec((BLK, D), lambda i: (i, 0)),
        out_shape=jax.ShapeDtypeStruct((NPAD, D), jnp.float32),
    )(p, nd, ns, b1, W2)


def _final_call(p, nd, b2):
    def body(p_ref, nd_ref, b_ref, o_ref):
        o_ref[...] = (p_ref[0] + p_ref[1]) * nd_ref[...] + b_ref[...]

    return pl.pallas_call(
        body,
        grid=(GRID,),
        in_specs=[
            pl.BlockSpec((NC, BLK, D), lambda i: (0, i, 0)),
            pl.BlockSpec((BLK, 1), lambda i: (i, 0)),
            pl.BlockSpec((1, D), lambda i: (0, 0)),
        ],
        out_specs=pl.BlockSpec((BLK, D), lambda i: (i, 0)),
        out_shape=jax.ShapeDtypeStruct((NPAD, D), jnp.float32),
    )(p, nd, b2)


def kernel(feat, edge_index, W1, b1, W2, b2):
    feat_p = jnp.pad(feat, ((0, NPAD - N), (0, 0)))
    er = edge_index.reshape(2, NW, EPT)
    er = jnp.pad(er, ((0, 0), (0, 0), (0, EPT_PAD - EPT)),
                 constant_values=PADROW)
    er = er.reshape(2, NW, NCH, CHUNK)
    src = er[0]
    dst = er[1]

    degp = _deg_call(src, dst).reshape(NC, 2, NPAD, 1)
    h1, ns, nd = _norms_matmul_call(degp, feat_p, W1)
    p1 = _agg_call(h1, src, dst)
    h2 = _mid_call(p1, nd, ns, b1.reshape(1, D), W2)
    p2 = _agg_call(h2, src, dst)
    out = _final_call(p2, nd, b2.reshape(1, D))
    return out[:N]


# speedup vs baseline: 11.0221x; 2.1440x over previous
import functools

import jax
import jax.numpy as jnp
from jax import lax
from jax.experimental import pallas as pl
from jax.experimental.pallas import tpu as pltpu
from jax.experimental.pallas import tpu_sc as plsc

N = 10000
D = 128
E = 320000

NC = 2
NS = 16
NW = NC * NS

NPAD = 10240
PADROW = NPAD - 1
EPT = E // NW
CHUNK = 80
NCH = 128
EPT_PAD = NCH * CHUNK
IB = 4
NBLK = NCH // IB
RPT = NPAD // NS
DEGW = 16

BLK = 1024
GRID = NPAD // BLK


def _mesh():
    return plsc.VectorSubcoreMesh(core_axis_name="c", subcore_axis_name="s")


def _deg_call(src, dst):
    @functools.partial(
        pl.kernel,
        mesh=_mesh(),
        out_type=jax.ShapeDtypeStruct((NC, 2, NPAD), jnp.float32),
        scratch_types=[
            pltpu.VMEM_SHARED((NPAD,), jnp.float32),
            pltpu.VMEM_SHARED((NPAD,), jnp.float32),
            pltpu.VMEM((NCH, CHUNK), jnp.int32),
            pltpu.VMEM((NCH, CHUNK), jnp.int32),
            pltpu.VMEM((CHUNK,), jnp.float32),
            pltpu.VMEM((RPT,), jnp.float32),
            pltpu.SemaphoreType.DMA,
        ],
    )
    def k(src_hbm, dst_hbm, out_hbm, acc_s, acc_d, isrc, idst, ones, zbuf,
          sem_sc):
        c = lax.axis_index("c")
        s = lax.axis_index("s")
        wid = s * NC + c
        off = s * RPT

        def fill_ones(r, _):
            ones[pl.ds(r * 16, 16)] = jnp.ones((16,), jnp.float32)
            return 0

        lax.fori_loop(0, CHUNK // 16, fill_ones, 0)

        def fill_zero(r, _):
            zbuf[pl.ds(r * 16, 16)] = jnp.zeros((16,), jnp.float32)
            return 0

        lax.fori_loop(0, RPT // 16, fill_zero, 0)

        pltpu.sync_copy(zbuf, acc_s.at[pl.ds(off, RPT)])
        pltpu.sync_copy(zbuf, acc_d.at[pl.ds(off, RPT)])
        pltpu.sync_copy(src_hbm.at[wid], isrc)
        pltpu.sync_copy(dst_hbm.at[wid], idst)
        plsc.subcore_barrier()

        def body(j, _):
            pltpu.async_copy(ones, acc_s.at[isrc.at[j]], sem_sc, add=True)
            pltpu.async_copy(ones, acc_d.at[idst.at[j]], sem_sc, add=True)
            return 0

        lax.fori_loop(0, NCH, body, 0)

        def drain(j, _):
            pltpu.make_async_copy(ones, acc_s.at[isrc.at[j]], sem_sc).wait()
            pltpu.make_async_copy(ones, acc_d.at[idst.at[j]], sem_sc).wait()
            return 0

        lax.fori_loop(0, NCH, drain, 0)
        plsc.subcore_barrier()
        pltpu.sync_copy(acc_s.at[pl.ds(off, RPT)], zbuf)
        pltpu.sync_copy(zbuf, out_hbm.at[c, 0, pl.ds(off, RPT)])
        pltpu.sync_copy(acc_d.at[pl.ds(off, RPT)], zbuf)
        pltpu.sync_copy(zbuf, out_hbm.at[c, 1, pl.ds(off, RPT)])

    return k(src, dst)


def _agg_call(h, src, dst):
    @functools.partial(
        pl.kernel,
        mesh=_mesh(),
        out_type=jax.ShapeDtypeStruct((NC, NPAD, D), jnp.float32),
        scratch_types=[
            pltpu.VMEM_SHARED((NPAD, D), jnp.float32),
            pltpu.VMEM((3, IB, CHUNK), jnp.int32),
            pltpu.VMEM((3, IB, CHUNK), jnp.int32),
            pltpu.VMEM((4, CHUNK, D), jnp.float32),
            pltpu.SemaphoreType.DMA((4,)),
            pltpu.SemaphoreType.DMA((4,)),
            pltpu.SemaphoreType.DMA((3,)),
            pltpu.SemaphoreType.DMA((3,)),
        ],
    )
    def k(h_hbm, src_hbm, dst_hbm, out_hbm, acc, isrc, idst, bufs,
          sem_g, sem_s, sem_is, sem_id):
        c = lax.axis_index("c")
        s = lax.axis_index("s")
        wid = s * NC + c
        off = s * RPT

        def fill_zero(i, _):
            r = i // (D // 16)
            col = (i % (D // 16)) * 16
            bufs[0, r, pl.ds(col, 16)] = jnp.zeros((16,), jnp.float32)
            return 0

        lax.fori_loop(0, CHUNK * (D // 16), fill_zero, 0)
        for t in range(RPT // CHUNK):
            pltpu.sync_copy(bufs.at[0], acc.at[pl.ds(off + t * CHUNK, CHUNK)])
        pltpu.sync_copy(src_hbm.at[wid, pl.ds(0, IB)], isrc.at[0])
        pltpu.sync_copy(dst_hbm.at[wid, pl.ds(0, IB)], idst.at[0])
        pltpu.async_copy(src_hbm.at[wid, pl.ds(IB, IB)], isrc.at[1],
                         sem_is.at[1])
        pltpu.async_copy(dst_hbm.at[wid, pl.ds(IB, IB)], idst.at[1],
                         sem_id.at[1])
        plsc.subcore_barrier()
        pltpu.async_copy(h_hbm.at[isrc.at[0, 0]], bufs.at[0], sem_g.at[0])
        pltpu.async_copy(h_hbm.at[isrc.at[0, 1]], bufs.at[1], sem_g.at[1])

        def body(b, _):
            h0 = lax.rem(b, 3)
            h1x = lax.rem(b + 1, 3)
            h2x = lax.rem(b + 2, 3)
            for r in range(IB):
                pltpu.make_async_copy(h_hbm.at[isrc.at[h0, r]], bufs.at[r],
                                      sem_g.at[r]).wait()
                pltpu.async_copy(bufs.at[r], acc.at[idst.at[h0, r]],
                                 sem_s.at[r], add=True)
                if r == IB - 2:
                    pltpu.make_async_copy(
                        src_hbm.at[wid, pl.ds((b + 1) * IB, IB)],
                        isrc.at[h1x], sem_is.at[h1x]).wait()
                    pltpu.make_async_copy(
                        dst_hbm.at[wid, pl.ds((b + 1) * IB, IB)],
                        idst.at[h1x], sem_id.at[h1x]).wait()
                if r >= 2:
                    pltpu.make_async_copy(bufs.at[r - 2],
                                          acc.at[idst.at[h0, r - 2]],
                                          sem_s.at[r - 2]).wait()
                    pltpu.async_copy(h_hbm.at[isrc.at[h1x, r - 2]],
                                     bufs.at[r - 2], sem_g.at[r - 2])
                else:
                    @pl.when(b > 0)
                    def _():
                        pltpu.make_async_copy(bufs.at[r + 2],
                                              acc.at[idst.at[h2x, r + 2]],
                                              sem_s.at[r + 2]).wait()

                    pltpu.async_copy(h_hbm.at[isrc.at[h0, r + 2]],
                                     bufs.at[r + 2], sem_g.at[r + 2])

            @pl.when(b + 2 < NBLK)
            def _():
                pltpu.async_copy(src_hbm.at[wid, pl.ds((b + 2) * IB, IB)],
                                 isrc.at[h2x], sem_is.at[h2x])
                pltpu.async_copy(dst_hbm.at[wid, pl.ds((b + 2) * IB, IB)],
                                 idst.at[h2x], sem_id.at[h2x])

            return 0

        lax.fori_loop(0, NBLK - 1, body, 0)
        s1 = (NBLK - 1) % 3
        s0 = (NBLK - 2) % 3
        for r in range(IB):
            pltpu.make_async_copy(h_hbm.at[isrc.at[s1, r]], bufs.at[r],
                                  sem_g.at[r]).wait()
            pltpu.async_copy(bufs.at[r], acc.at[idst.at[s1, r]],
                             sem_s.at[r], add=True)
            if r >= 2:
                pltpu.make_async_copy(bufs.at[r - 2],
                                      acc.at[idst.at[s1, r - 2]],
                                      sem_s.at[r - 2]).wait()
            else:
                pltpu.make_async_copy(bufs.at[r + 2],
                                      acc.at[idst.at[s0, r + 2]],
                                      sem_s.at[r + 2]).wait()
                pltpu.async_copy(h_hbm.at[isrc.at[s1, r + 2]],
                                 bufs.at[r + 2], sem_g.at[r + 2])
        pltpu.make_async_copy(bufs.at[2], acc.at[idst.at[s1, 2]],
                              sem_s.at[2]).wait()
        pltpu.make_async_copy(bufs.at[3], acc.at[idst.at[s1, 3]],
                              sem_s.at[3]).wait()
        plsc.subcore_barrier()

        def copy_out(t, _):
            pltpu.sync_copy(acc.at[pl.ds(off + t * CHUNK, CHUNK)], bufs.at[0])
            pltpu.sync_copy(bufs.at[0],
                            out_hbm.at[c, pl.ds(off + t * CHUNK, CHUNK)])
            return 0

        lax.fori_loop(0, RPT // CHUNK, copy_out, 0)

    return k(h, src, dst)


def _matmul_call(x, W):
    def body(x_ref, w_ref, o_ref):
        o_ref[...] = jnp.dot(x_ref[...], w_ref[...],
                             preferred_element_type=jnp.float32)

    return pl.pallas_call(
        body,
        grid=(GRID,),
        in_specs=[
            pl.BlockSpec((BLK, D), lambda i: (i, 0)),
            pl.BlockSpec((D, D), lambda i: (0, 0)),
        ],
        out_specs=pl.BlockSpec((BLK, D), lambda i: (i, 0)),
        out_shape=jax.ShapeDtypeStruct((NPAD, D), jnp.float32),
    )(x, W)


def _norms_scale_call(degp, xw):
    def body(degp_ref, xw_ref, h_ref, ns_ref, nd_ref):
        x = degp_ref[...]
        d_o = x[0, 0, :, 0:1] + x[1, 0, :, 0:1]
        d_i = x[0, 1, :, 0:1] + x[1, 1, :, 0:1]
        ns = lax.rsqrt(jnp.where(d_o > 0, d_o, 1.0))
        nd = lax.rsqrt(jnp.where(d_i > 0, d_i, 1.0))
        ns_ref[...] = ns
        nd_ref[...] = nd
        h_ref[...] = xw_ref[...] * ns

    return pl.pallas_call(
        body,
        grid=(GRID,),
        in_specs=[
            pl.BlockSpec((NC, 2, BLK, 1), lambda i: (0, 0, i, 0)),
            pl.BlockSpec((BLK, D), lambda i: (i, 0)),
        ],
        out_specs=[
            pl.BlockSpec((BLK, D), lambda i: (i, 0)),
            pl.BlockSpec((BLK, 1), lambda i: (i, 0)),
            pl.BlockSpec((BLK, 1), lambda i: (i, 0)),
        ],
        out_shape=[
            jax.ShapeDtypeStruct((NPAD, D), jnp.float32),
            jax.ShapeDtypeStruct((NPAD, 1), jnp.float32),
            jax.ShapeDtypeStruct((NPAD, 1), jnp.float32),
        ],
    )(degp, xw)


def _mid_call(p, nd, ns, b1, W2):
    def body(p_ref, nd_ref, ns_ref, b_ref, w_ref, h_ref):
        agg = p_ref[0] + p_ref[1]
        o = jnp.maximum(agg * nd_ref[...] + b_ref[...], 0.0)
        h_ref[...] = jnp.dot(o * ns_ref[...], w_ref[...],
                             preferred_element_type=jnp.float32)

    return pl.pallas_call(
        body,
        grid=(GRID,),
        in_specs=[
            pl.BlockSpec((NC, BLK, D), lambda i: (0, i, 0)),
            pl.BlockSpec((BLK, 1), lambda i: (i, 0)),
            pl.BlockSpec((BLK, 1), lambda i: (i, 0)),
            pl.BlockSpec((1, D), lambda i: (0, 0)),
            pl.BlockSpec((D, D), lambda i: (0, 0)),
        ],
        out_specs=pl.BlockSpec((BLK, D), lambda i: (i, 0)),
        out_shape=jax.ShapeDtypeStruct((NPAD, D), jnp.float32),
    )(p, nd, ns, b1, W2)


def _final_call(p, nd, b2):
    def body(p_ref, nd_ref, b_ref, o_ref):
        o_ref[...] = (p_ref[0] + p_ref[1]) * nd_ref[...] + b_ref[...]

    return pl.pallas_call(
        body,
        grid=(GRID,),
        in_specs=[
            pl.BlockSpec((NC, BLK, D), lambda i: (0, i, 0)),
            pl.BlockSpec((BLK, 1), lambda i: (i, 0)),
            pl.BlockSpec((1, D), lambda i: (0, 0)),
        ],
        out_specs=pl.BlockSpec((BLK, D), lambda i: (i, 0)),
        out_shape=jax.ShapeDtypeStruct((NPAD, D), jnp.float32),
    )(p, nd, b2)


def kernel(feat, edge_index, W1, b1, W2, b2):
    feat_p = jnp.pad(feat, ((0, NPAD - N), (0, 0)))
    er = edge_index.reshape(2, NW, EPT)
    ramp = N + (jnp.arange(EPT_PAD - EPT, dtype=jnp.int32) % (NPAD - N))
    er = jnp.concatenate(
        [er, jnp.broadcast_to(ramp, (2, NW, EPT_PAD - EPT))], axis=2)
    er = er.reshape(2, NW, NCH, CHUNK)
    src = er[0]
    dst = er[1]

    xw = _matmul_call(feat_p, W1)
    degp = _deg_call(src, dst).reshape(NC, 2, NPAD, 1)
    h1, ns, nd = _norms_scale_call(degp, xw)
    p1 = _agg_call(h1, src, dst)
    h2 = _mid_call(p1, nd, ns, b1.reshape(1, D), W2)
    p2 = _agg_call(h2, src, dst)
    out = _final_call(p2, nd, b2.reshape(1, D))
    return out[:N]


# speedup vs baseline: 11.8240x; 1.0728x over previous
import functools

import jax
import jax.numpy as jnp
from jax import lax
from jax.experimental import pallas as pl
from jax.experimental.pallas import tpu as pltpu
from jax.experimental.pallas import tpu_sc as plsc

N = 10000
D = 128
E = 320000

NC = 2
NS = 16
NW = NC * NS

NPAD = 10240
PADROW = NPAD - 1
EPT = E // NW
CHUNK = 128
NCH = -(-EPT // CHUNK)
NCH = NCH + (NCH % 2)
EPT_PAD = NCH * CHUNK
IB = 4
NBLK = NCH // IB
RPT = NPAD // NS
DEGW = 16

BLK = 1024
GRID = NPAD // BLK


def _mesh():
    return plsc.VectorSubcoreMesh(core_axis_name="c", subcore_axis_name="s")


def _deg_call(src, dst):
    @functools.partial(
        pl.kernel,
        mesh=_mesh(),
        out_type=jax.ShapeDtypeStruct((NC, 2, NPAD), jnp.float32),
        scratch_types=[
            pltpu.VMEM_SHARED((NPAD,), jnp.float32),
            pltpu.VMEM_SHARED((NPAD,), jnp.float32),
            pltpu.VMEM((NCH, CHUNK), jnp.int32),
            pltpu.VMEM((NCH, CHUNK), jnp.int32),
            pltpu.VMEM((CHUNK,), jnp.float32),
            pltpu.VMEM((RPT,), jnp.float32),
            pltpu.SemaphoreType.DMA,
        ],
    )
    def k(src_hbm, dst_hbm, out_hbm, acc_s, acc_d, isrc, idst, ones, zbuf,
          sem_sc):
        c = lax.axis_index("c")
        s = lax.axis_index("s")
        wid = s * NC + c
        off = s * RPT

        def fill_ones(r, _):
            ones[pl.ds(r * 16, 16)] = jnp.ones((16,), jnp.float32)
            return 0

        lax.fori_loop(0, CHUNK // 16, fill_ones, 0)

        def fill_zero(r, _):
            zbuf[pl.ds(r * 16, 16)] = jnp.zeros((16,), jnp.float32)
            return 0

        lax.fori_loop(0, RPT // 16, fill_zero, 0)

        pltpu.sync_copy(zbuf, acc_s.at[pl.ds(off, RPT)])
        pltpu.sync_copy(zbuf, acc_d.at[pl.ds(off, RPT)])
        pltpu.sync_copy(src_hbm.at[wid], isrc)
        pltpu.sync_copy(dst_hbm.at[wid], idst)
        plsc.subcore_barrier()

        def body(j, _):
            pltpu.async_copy(ones, acc_s.at[isrc.at[j]], sem_sc, add=True)
            pltpu.async_copy(ones, acc_d.at[idst.at[j]], sem_sc, add=True)
            return 0

        lax.fori_loop(0, NCH, body, 0)

        def drain(j, _):
            pltpu.make_async_copy(ones, acc_s.at[isrc.at[j]], sem_sc).wait()
            pltpu.make_async_copy(ones, acc_d.at[idst.at[j]], sem_sc).wait()
            return 0

        lax.fori_loop(0, NCH, drain, 0)
        plsc.subcore_barrier()
        pltpu.sync_copy(acc_s.at[pl.ds(off, RPT)], zbuf)
        pltpu.sync_copy(zbuf, out_hbm.at[c, 0, pl.ds(off, RPT)])
        pltpu.sync_copy(acc_d.at[pl.ds(off, RPT)], zbuf)
        pltpu.sync_copy(zbuf, out_hbm.at[c, 1, pl.ds(off, RPT)])

    return k(src, dst)


def _agg_call(h, src, dst):
    @functools.partial(
        pl.kernel,
        mesh=_mesh(),
        out_type=jax.ShapeDtypeStruct((NC, NPAD, D), jnp.float32),
        scratch_types=[
            pltpu.VMEM_SHARED((NPAD, D), jnp.float32),
            pltpu.VMEM((2, IB, CHUNK), jnp.int32),
            pltpu.VMEM((NCH, CHUNK), jnp.int32),
            pltpu.VMEM((CHUNK, D), jnp.float32),
            pltpu.VMEM((CHUNK, D), jnp.float32),
            pltpu.SemaphoreType.DMA((2,)),
            pltpu.SemaphoreType.DMA((2,)),
        ],
    )
    def k(h_hbm, src_hbm, dst_hbm, out_hbm, acc, isrc, idst, b0, b1,
          sem_g, sem_is):
        c = lax.axis_index("c")
        s = lax.axis_index("s")
        wid = s * NC + c
        off = s * RPT
        bufs01 = (b0, b1)

        def fill_zero(i, _):
            r = i // (D // 16)
            col = (i % (D // 16)) * 16
            b0[r, pl.ds(col, 16)] = jnp.zeros((16,), jnp.float32)
            return 0

        lax.fori_loop(0, CHUNK * (D // 16), fill_zero, 0)
        for t in range(RPT // CHUNK):
            pltpu.sync_copy(b0, acc.at[pl.ds(off + t * CHUNK, CHUNK)])
        pltpu.sync_copy(dst_hbm.at[wid], idst)
        pltpu.sync_copy(src_hbm.at[wid, pl.ds(0, IB)], isrc.at[0])
        pltpu.async_copy(src_hbm.at[wid, pl.ds(IB, IB)], isrc.at[1],
                         sem_is.at[1])
        plsc.subcore_barrier()
        pltpu.async_copy(h_hbm.at[isrc.at[0, 0]], b0, sem_g.at[0])
        pltpu.async_copy(h_hbm.at[isrc.at[0, 1]], b1, sem_g.at[1])

        def body(b, _):
            hb = lax.rem(b, 2)
            hb1 = 1 - hb
            jbase = b * IB
            for r in range(IB):
                buf = bufs01[r % 2]
                gsem = sem_g.at[r % 2]
                pltpu.make_async_copy(h_hbm.at[isrc.at[hb, r]], buf,
                                      gsem).wait()
                pltpu.sync_copy(buf, acc.at[idst.at[jbase + r]], add=True)
                if r == IB - 2:
                    pltpu.make_async_copy(
                        src_hbm.at[wid, pl.ds((b + 1) * IB, IB)],
                        isrc.at[hb1], sem_is.at[hb1]).wait()
                if r < IB - 2:
                    pltpu.async_copy(h_hbm.at[isrc.at[hb, r + 2]], buf, gsem)
                else:
                    pltpu.async_copy(h_hbm.at[isrc.at[hb1, r - (IB - 2)]],
                                     buf, gsem)

            @pl.when(b + 2 < NBLK)
            def _():
                pltpu.async_copy(src_hbm.at[wid, pl.ds((b + 2) * IB, IB)],
                                 isrc.at[hb], sem_is.at[hb])

            return 0

        lax.fori_loop(0, NBLK - 1, body, 0)
        lastslot = (NBLK - 1) % 2
        for r in range(IB):
            buf = bufs01[r % 2]
            gsem = sem_g.at[r % 2]
            pltpu.make_async_copy(h_hbm.at[isrc.at[lastslot, r]], buf,
                                  gsem).wait()
            pltpu.sync_copy(buf, acc.at[idst.at[(NBLK - 1) * IB + r]],
                            add=True)
            if r < IB - 2:
                pltpu.async_copy(h_hbm.at[isrc.at[lastslot, r + 2]], buf,
                                 gsem)
        plsc.subcore_barrier()

        def copy_out(t, _):
            pltpu.sync_copy(acc.at[pl.ds(off + t * CHUNK, CHUNK)], b0)
            pltpu.sync_copy(b0, out_hbm.at[c, pl.ds(off + t * CHUNK, CHUNK)])
            return 0

        lax.fori_loop(0, RPT // CHUNK, copy_out, 0)

    return k(h, src, dst)


def _matmul_call(x, W):
    def body(x_ref, w_ref, o_ref):
        o_ref[...] = jnp.dot(x_ref[...], w_ref[...],
                             preferred_element_type=jnp.float32)

    return pl.pallas_call(
        body,
        grid=(GRID,),
        in_specs=[
            pl.BlockSpec((BLK, D), lambda i: (i, 0)),
            pl.BlockSpec((D, D), lambda i: (0, 0)),
        ],
        out_specs=pl.BlockSpec((BLK, D), lambda i: (i, 0)),
        out_shape=jax.ShapeDtypeStruct((NPAD, D), jnp.float32),
    )(x, W)


def _norms_scale_call(degp, xw):
    def body(degp_ref, xw_ref, h_ref, ns_ref, nd_ref):
        x = degp_ref[...]
        d_o = x[0, 0, :, 0:1] + x[1, 0, :, 0:1]
        d_i = x[0, 1, :, 0:1] + x[1, 1, :, 0:1]
        ns = lax.rsqrt(jnp.where(d_o > 0, d_o, 1.0))
        nd = lax.rsqrt(jnp.where(d_i > 0, d_i, 1.0))
        ns_ref[...] = ns
        nd_ref[...] = nd
        h_ref[...] = xw_ref[...] * ns

    return pl.pallas_call(
        body,
        grid=(GRID,),
        in_specs=[
            pl.BlockSpec((NC, 2, BLK, 1), lambda i: (0, 0, i, 0)),
            pl.BlockSpec((BLK, D), lambda i: (i, 0)),
        ],
        out_specs=[
            pl.BlockSpec((BLK, D), lambda i: (i, 0)),
            pl.BlockSpec((BLK, 1), lambda i: (i, 0)),
            pl.BlockSpec((BLK, 1), lambda i: (i, 0)),
        ],
        out_shape=[
            jax.ShapeDtypeStruct((NPAD, D), jnp.float32),
            jax.ShapeDtypeStruct((NPAD, 1), jnp.float32),
            jax.ShapeDtypeStruct((NPAD, 1), jnp.float32),
        ],
    )(degp, xw)


def _mid_call(p, nd, ns, b1, W2):
    def body(p_ref, nd_ref, ns_ref, b_ref, w_ref, h_ref):
        agg = p_ref[0] + p_ref[1]
        o = jnp.maximum(agg * nd_ref[...] + b_ref[...], 0.0)
        h_ref[...] = jnp.dot(o * ns_ref[...], w_ref[...],
                             preferred_element_type=jnp.float32)

    return pl.pallas_call(
        body,
        grid=(GRID,),
        in_specs=[
            pl.BlockSpec((NC, BLK, D), lambda i: (0, i, 0)),
            pl.BlockSpec((BLK, 1), lambda i: (i, 0)),
            pl.BlockSpec((BLK, 1), lambda i: (i, 0)),
            pl.BlockSpec((1, D), lambda i: (0, 0)),
            pl.BlockSpec((D, D), lambda i: (0, 0)),
        ],
        out_specs=pl.BlockSpec((BLK, D), lambda i: (i, 0)),
        out_shape=jax.ShapeDtypeStruct((NPAD, D), jnp.float32),
    )(p, nd, ns, b1, W2)


def _final_call(p, nd, b2):
    def body(p_ref, nd_ref, b_ref, o_ref):
        o_ref[...] = (p_ref[0] + p_ref[1]) * nd_ref[...] + b_ref[...]

    return pl.pallas_call(
        body,
        grid=(GRID,),
        in_specs=[
            pl.BlockSpec((NC, BLK, D), lambda i: (0, i, 0)),
            pl.BlockSpec((BLK, 1), lambda i: (i, 0)),
            pl.BlockSpec((1, D), lambda i: (0, 0)),
        ],
        out_specs=pl.BlockSpec((BLK, D), lambda i: (i, 0)),
        out_shape=jax.ShapeDtypeStruct((NPAD, D), jnp.float32),
    )(p, nd, b2)


def kernel(feat, edge_index, W1, b1, W2, b2):
    feat_p = jnp.pad(feat, ((0, NPAD - N), (0, 0)))
    er = edge_index.reshape(2, NW, EPT)
    ramp = N + (jnp.arange(EPT_PAD - EPT, dtype=jnp.int32) % (NPAD - N))
    er = jnp.concatenate(
        [er, jnp.broadcast_to(ramp, (2, NW, EPT_PAD - EPT))], axis=2)
    er = er.reshape(2, NW, NCH, CHUNK)
    src = er[0]
    dst = er[1]

    xw = _matmul_call(feat_p, W1)
    degp = _deg_call(src, dst).reshape(NC, 2, NPAD, 1)
    h1, ns, nd = _norms_scale_call(degp, xw)
    p1 = _agg_call(h1, src, dst)
    h2 = _mid_call(p1, nd, ns, b1.reshape(1, D), W2)
    p2 = _agg_call(h2, src, dst)
    out = _final_call(p2, nd, b2.reshape(1, D))
    return out[:N]
